# Initial kernel scaffold; baseline (speedup 1.0000x reference)
import jax
import jax.numpy as jnp
from jax.experimental import pallas as pl


def kernel(H, H_case, H_disease, X, Theta):
    raise NotImplementedError("write your pallas kernel here")


# speedup vs baseline: 1.8628x; 1.8628x over previous
import jax
import jax.numpy as jnp
from jax.experimental import pallas as pl
from jax.experimental.pallas import tpu as pltpu


def _p1_body(nbr, h_ref, x_ref, edge_ref, de_ref, acc_ref, dacc_ref):
    i = pl.program_id(0)
    h = h_ref[...]
    hb = h.astype(jnp.bfloat16)
    xb = x_ref[...].astype(jnp.bfloat16)
    part = jax.lax.dot_general(hb, xb, (((0,), (0,)), ((), ())),
                               preferred_element_type=jnp.float32)
    ones = jnp.ones((h.shape[0], 8), jnp.bfloat16)
    dpart = jax.lax.dot_general(hb, ones, (((0,), (0,)), ((), ())),
                                preferred_element_type=jnp.float32)

    @pl.when(i == 0)
    def _():
        acc_ref[...] = part
        dacc_ref[...] = dpart

    @pl.when(i > 0)
    def _():
        acc_ref[...] += part
        dacc_ref[...] += dpart

    @pl.when(i == nbr - 1)
    def _():
        de = jnp.maximum(dacc_ref[...][:, 0:1], 1e-6)
        edge_ref[...] = (acc_ref[...] / de).astype(jnp.bfloat16)
        de_ref[...] = de


def _p2_body(nbr, h_ref, edge_ref, theta_ref, repr_ref, acc_ref):
    i = pl.program_id(0)
    h = h_ref[...]
    hb = h.astype(jnp.bfloat16)
    node = jnp.dot(hb, edge_ref[...], preferred_element_type=jnp.float32)
    ones = jnp.ones((h.shape[1], 8), jnp.bfloat16)
    dv = jax.lax.dot_general(hb, ones, (((1,), (0,)), ((), ())),
                             preferred_element_type=jnp.float32)
    node = node / jnp.maximum(dv[:, 0:1], 1e-6)
    nr = jnp.dot(node.astype(jnp.bfloat16), theta_ref[...].astype(jnp.bfloat16),
                 preferred_element_type=jnp.float32)
    nr = jnp.maximum(nr, 0.0)
    contrib = jax.lax.dot_general(hb, nr.astype(jnp.bfloat16),
                                  (((0,), (0,)), ((), ())),
                                  preferred_element_type=jnp.float32)

    @pl.when(i == 0)
    def _():
        acc_ref[...] = contrib

    @pl.when(i > 0)
    def _():
        acc_ref[...] += contrib

    @pl.when(i == nbr - 1)
    def _():
        repr_ref[...] = acc_ref[...]


def _p3_body(case_ref, dis_ref, decase_ref, dedis_ref, scores_ref, glob_ref):
    j = pl.program_id(0)
    c = case_ref[...] / jnp.maximum(decase_ref[...], 1e-6)
    cn = c / jnp.maximum(jnp.sqrt(jnp.sum(c * c, axis=1, keepdims=True)), 1e-8)
    d = dis_ref[...] / jnp.maximum(dedis_ref[...], 1e-6)
    dn = d / jnp.maximum(jnp.sqrt(jnp.sum(d * d, axis=1, keepdims=True)), 1e-8)
    s = jax.lax.dot_general(cn, dn, (((1,), (1,)), ((), ())),
                            preferred_element_type=jnp.float32)
    scores_ref[...] = s

    @pl.when(j == 0)
    def _():
        glob_ref[...] = jnp.full(glob_ref.shape, jnp.finfo(jnp.float32).min,
                                 jnp.float32)

    @pl.when(j > 0)
    def _():
        glob_ref[...] = s


def kernel(H, H_case, H_disease, X, Theta):
    n_hpo, n_edges = H.shape
    n_case = H_case.shape[1]
    n_disease = H_disease.shape[1]
    hidden = X.shape[1]

    br = 400 if n_hpo % 400 == 0 else n_hpo
    nbr = n_hpo // br
    cb = 1024 if (n_case % 1024 == 0 and n_disease % 1024 == 0) else n_case
    ngrid3 = 1 + n_disease // cb

    import functools
    edge, de = pl.pallas_call(
        functools.partial(_p1_body, nbr),
        grid=(nbr,),
        in_specs=[
            pl.BlockSpec((br, n_edges), lambda i: (i, 0)),
            pl.BlockSpec((br, hidden), lambda i: (i, 0)),
        ],
        out_specs=[
            pl.BlockSpec((n_edges, hidden), lambda i: (0, 0)),
            pl.BlockSpec((n_edges, 1), lambda i: (0, 0)),
        ],
        out_shape=[
            jax.ShapeDtypeStruct((n_edges, hidden), jnp.bfloat16),
            jax.ShapeDtypeStruct((n_edges, 1), jnp.float32),
        ],
        scratch_shapes=[
            pltpu.VMEM((n_edges, hidden), jnp.float32),
            pltpu.VMEM((n_edges, 8), jnp.float32),
        ],
    )(H, X)

    repr_ = pl.pallas_call(
        functools.partial(_p2_body, nbr),
        grid=(nbr,),
        in_specs=[
            pl.BlockSpec((br, n_edges), lambda i: (i, 0)),
            pl.BlockSpec((n_edges, hidden), lambda i: (0, 0)),
            pl.BlockSpec((hidden, hidden), lambda i: (0, 0)),
        ],
        out_specs=pl.BlockSpec((n_edges, hidden), lambda i: (0, 0)),
        out_shape=jax.ShapeDtypeStruct((n_edges, hidden), jnp.float32),
        scratch_shapes=[
            pltpu.VMEM((n_edges, hidden), jnp.float32),
        ],
    )(H, edge, Theta)

    scores, scores_global = pl.pallas_call(
        _p3_body,
        grid=(ngrid3,),
        in_specs=[
            pl.BlockSpec((n_case, hidden), lambda j: (0, 0)),
            pl.BlockSpec((cb, hidden), lambda j: (jnp.maximum(j, 1), 0)),
            pl.BlockSpec((n_case, 1), lambda j: (0, 0)),
            pl.BlockSpec((cb, 1), lambda j: (jnp.maximum(j, 1), 0)),
        ],
        out_specs=[
            pl.BlockSpec((n_case, cb), lambda j: (0, jnp.maximum(j, 1) - 1)),
            pl.BlockSpec((n_case, cb), lambda j: (0, j)),
        ],
        out_shape=[
            jax.ShapeDtypeStruct((n_case, n_disease), jnp.float32),
            jax.ShapeDtypeStruct((n_case, n_case + n_disease), jnp.float32),
        ],
    )(repr_, repr_, de, de)
    return scores, scores_global


# speedup vs baseline: 2.4434x; 1.3116x over previous
import functools

import jax
import jax.numpy as jnp
from jax.experimental import pallas as pl
from jax.experimental.pallas import tpu as pltpu


def _p1_body(nbr, hid, h_ref, xa_ref, edge_ref, de_ref, acc_ref):
    i = pl.program_id(0)
    hb = h_ref[...].astype(jnp.bfloat16)
    xab = xa_ref[...].astype(jnp.bfloat16)
    part = jax.lax.dot_general(xab, hb, (((0,), (0,)), ((), ())),
                               preferred_element_type=jnp.float32)

    @pl.when(i == 0)
    def _():
        acc_ref[...] = part

    @pl.when(i > 0)
    def _():
        acc_ref[...] += part

    @pl.when(i == nbr - 1)
    def _():
        de = jnp.maximum(acc_ref[hid:hid + 1, :], 1e-6)
        edge_ref[0:hid, :] = (acc_ref[0:hid, :] / de).astype(jnp.bfloat16)
        edge_ref[hid:, :] = jnp.ones_like(edge_ref[hid:, :])
        de_ref[...] = de


def _p2_body(nbr, hid, h_ref, edge_ref, theta_ref, repr_ref, acc_ref):
    i = pl.program_id(0)
    hb = h_ref[...].astype(jnp.bfloat16)
    node_aug = jax.lax.dot_general(edge_ref[...], hb, (((1,), (1,)), ((), ())),
                                   preferred_element_type=jnp.float32)
    dv = jnp.maximum(node_aug[hid:hid + 1, :], 1e-6)
    node_t = node_aug[0:hid, :] / dv
    nr_t = jax.lax.dot_general(theta_ref[...].astype(jnp.bfloat16),
                               node_t.astype(jnp.bfloat16),
                               (((0,), (0,)), ((), ())),
                               preferred_element_type=jnp.float32)
    nr_t = jnp.maximum(nr_t, 0.0).astype(jnp.bfloat16)
    contrib = jax.lax.dot_general(nr_t, hb, (((1,), (0,)), ((), ())),
                                  preferred_element_type=jnp.float32)

    @pl.when(i == 0)
    def _():
        acc_ref[...] = contrib

    @pl.when(i > 0)
    def _():
        acc_ref[...] += contrib

    @pl.when(i == nbr - 1)
    def _():
        repr_ref[...] = acc_ref[...]


def _p3_body(case_ref, dis_ref, decase_ref, dedis_ref, scores_ref, glob_ref):
    j = pl.program_id(0)
    c = case_ref[...] / jnp.maximum(decase_ref[...], 1e-6)
    cn = c / jnp.maximum(jnp.sqrt(jnp.sum(c * c, axis=0, keepdims=True)), 1e-8)
    d = dis_ref[...] / jnp.maximum(dedis_ref[...], 1e-6)
    dn = d / jnp.maximum(jnp.sqrt(jnp.sum(d * d, axis=0, keepdims=True)), 1e-8)
    s = jax.lax.dot_general(cn, dn, (((0,), (0,)), ((), ())),
                            preferred_element_type=jnp.float32)
    scores_ref[...] = s

    @pl.when(j == 0)
    def _():
        glob_ref[...] = jnp.full(glob_ref.shape, jnp.finfo(jnp.float32).min,
                                 jnp.float32)

    @pl.when(j > 0)
    def _():
        glob_ref[...] = s


def kernel(H, H_case, H_disease, X, Theta):
    n_hpo, n_edges = H.shape
    n_case = H_case.shape[1]
    n_disease = H_disease.shape[1]
    hid = X.shape[1]
    hid_a = hid + 8

    br = 400 if n_hpo % 400 == 0 else n_hpo
    nbr = n_hpo // br
    cb = 1024 if (n_case % 1024 == 0 and n_disease % 1024 == 0) else n_case
    ngrid3 = 1 + n_disease // cb

    Xa = jnp.concatenate(
        [X, jnp.ones((n_hpo, 1), X.dtype), jnp.zeros((n_hpo, 7), X.dtype)],
        axis=1)

    edge_aug, de = pl.pallas_call(
        functools.partial(_p1_body, nbr, hid),
        grid=(nbr,),
        in_specs=[
            pl.BlockSpec((br, n_edges), lambda i: (i, 0)),
            pl.BlockSpec((br, hid_a), lambda i: (i, 0)),
        ],
        out_specs=[
            pl.BlockSpec((hid_a, n_edges), lambda i: (0, 0)),
            pl.BlockSpec((1, n_edges), lambda i: (0, 0)),
        ],
        out_shape=[
            jax.ShapeDtypeStruct((hid_a, n_edges), jnp.bfloat16),
            jax.ShapeDtypeStruct((1, n_edges), jnp.float32),
        ],
        scratch_shapes=[
            pltpu.VMEM((hid_a, n_edges), jnp.float32),
        ],
    )(H, Xa)

    repr_t = pl.pallas_call(
        functools.partial(_p2_body, nbr, hid),
        grid=(nbr,),
        in_specs=[
            pl.BlockSpec((br, n_edges), lambda i: (i, 0)),
            pl.BlockSpec((hid_a, n_edges), lambda i: (0, 0)),
            pl.BlockSpec((hid, hid), lambda i: (0, 0)),
        ],
        out_specs=pl.BlockSpec((hid, n_edges), lambda i: (0, 0)),
        out_shape=jax.ShapeDtypeStruct((hid, n_edges), jnp.float32),
        scratch_shapes=[
            pltpu.VMEM((hid, n_edges), jnp.float32),
        ],
    )(H, edge_aug, Theta)

    scores, scores_global = pl.pallas_call(
        _p3_body,
        grid=(ngrid3,),
        in_specs=[
            pl.BlockSpec((hid, n_case), lambda j: (0, 0)),
            pl.BlockSpec((hid, cb), lambda j: (0, jnp.maximum(j, 1))),
            pl.BlockSpec((1, n_case), lambda j: (0, 0)),
            pl.BlockSpec((1, cb), lambda j: (0, jnp.maximum(j, 1))),
        ],
        out_specs=[
            pl.BlockSpec((n_case, cb), lambda j: (0, jnp.maximum(j, 1) - 1)),
            pl.BlockSpec((n_case, cb), lambda j: (0, j)),
        ],
        out_shape=[
            jax.ShapeDtypeStruct((n_case, n_disease), jnp.float32),
            jax.ShapeDtypeStruct((n_case, n_case + n_disease), jnp.float32),
        ],
    )(repr_t, repr_t, de, de)
    return scores, scores_global
